# Initial kernel scaffold; baseline (speedup 1.0000x reference)
import jax
import jax.numpy as jnp
from jax.experimental import pallas as pl


def kernel(output, target, target_weight):
    raise NotImplementedError("write your pallas kernel here")


# speedup vs baseline: 5.2088x; 5.2088x over previous
import functools

import jax
import jax.numpy as jnp
from jax import lax
from jax.experimental import pallas as pl
from jax.experimental.pallas import tpu as pltpu
from jax.experimental.pallas import tpu_sc as plsc

_TOPK = 5
_NC = 2
_NS = 16
_NW = _NC * _NS
_L = 16
_BL = 128

_NEG = float(jnp.finfo(jnp.float32).min)


@functools.lru_cache(maxsize=None)
def _build(batch: int, joints: int, interpret: bool = False):
    bg = batch // _BL
    rows_per_w = batch // _NW
    chunk = 64
    nchunk = rows_per_w // chunk
    ngrp = chunk // _L

    mesh = plsc.VectorSubcoreMesh(
        core_axis_name="c", subcore_axis_name="s", num_cores=_NC,
        num_subcores=_NS)

    @functools.partial(
        pl.kernel,
        out_type=jax.ShapeDtypeStruct((_NW * _L,), jnp.float32),
        mesh=mesh,
        scratch_types=[
            pltpu.VMEM((joints, 2, chunk), jnp.float32),
            pltpu.VMEM((joints, 2, chunk), jnp.float32),
            pltpu.VMEM((joints, chunk), jnp.float32),
            pltpu.VMEM((_L,), jnp.float32),
        ],
        compiler_params=pltpu.CompilerParams(
            use_tc_tiling_on_sc=False, needs_layout_passes=False),
        interpret=interpret,
    )
    def sc_kernel(o_hbm, t_hbm, w_hbm, out_hbm, o_v, t_v, w_v, acc_v):
        cid = lax.axis_index("c")
        sid = lax.axis_index("s")
        wid = sid * _NC + cid
        b0 = wid * rows_per_w
        acc_v[...] = jnp.zeros((_L,), jnp.float32)

        def chunk_body(ci, carry):
            b = b0 + ci * chunk
            g = b // _BL
            l0 = b % _BL
            pltpu.sync_copy(o_hbm.at[:, g, :, pl.ds(l0, chunk)], o_v)
            pltpu.sync_copy(t_hbm.at[:, g, :, pl.ds(l0, chunk)], t_v)
            pltpu.sync_copy(w_hbm.at[:, pl.ds(b, chunk)], w_v)

            neg = jnp.full((_L,), _NEG, jnp.float32)

            for gi in range(ngrp):
                s = gi * _L

                def jbody(j, ms):
                    m1, m2, m3, m4, m5 = ms
                    o0 = o_v[j, 0, pl.ds(s, _L)]
                    o1 = o_v[j, 1, pl.ds(s, _L)]
                    t0 = t_v[j, 0, pl.ds(s, _L)]
                    t1 = t_v[j, 1, pl.ds(s, _L)]
                    tw = w_v[j, pl.ds(s, _L)]
                    d0 = o0 - t0
                    d1 = o1 - t1
                    v = (d0 * d0 + d1 * d1) * tw
                    n1 = jnp.maximum(m1, v)
                    r = jnp.minimum(m1, v)
                    n2 = jnp.maximum(m2, r)
                    r = jnp.minimum(m2, r)
                    n3 = jnp.maximum(m3, r)
                    r = jnp.minimum(m3, r)
                    n4 = jnp.maximum(m4, r)
                    r = jnp.minimum(m4, r)
                    n5 = jnp.maximum(m5, r)
                    return (n1, n2, n3, n4, n5)

                m1, m2, m3, m4, m5 = lax.fori_loop(
                    0, joints, jbody, (neg, neg, neg, neg, neg))
                acc_v[...] = acc_v[...] + (m1 + m2 + m3 + m4 + m5)
            return carry

        lax.fori_loop(0, nchunk, chunk_body, 0)
        pltpu.sync_copy(acc_v, out_hbm.at[pl.ds(wid * _L, _L)])

    return sc_kernel


def kernel(output, target, target_weight):
    batch, joints, _ = output.shape
    o4 = output.reshape(_BL, batch // _BL, joints, 2).transpose(2, 0, 3, 1)
    t4 = target.reshape(_BL, batch // _BL, joints, 2).transpose(2, 0, 3, 1)
    wt = target_weight.T
    parts = _build(batch, joints)(o4, t4, wt)
    return jnp.sum(parts) * (1.0 / (_TOPK * batch))


# speedup vs baseline: 7.5883x; 1.4568x over previous
import functools

import jax
import jax.numpy as jnp
from jax import lax
from jax.experimental import pallas as pl
from jax.experimental.pallas import tpu as pltpu
from jax.experimental.pallas import tpu_sc as plsc

_TOPK = 5
_NC = 2
_NS = 16
_NW = _NC * _NS
_L = 16
_BL = 128

_NEG = float(jnp.finfo(jnp.float32).min)


@functools.lru_cache(maxsize=None)
def _build(batch: int, joints: int, interpret: bool = False):
    rows_per_w = batch // _NW
    chunk = 64
    nchunk = rows_per_w // chunk
    ngrp = chunk // _L

    mesh = plsc.VectorSubcoreMesh(
        core_axis_name="c", subcore_axis_name="s", num_cores=_NC,
        num_subcores=_NS)

    @functools.partial(
        pl.kernel,
        out_type=jax.ShapeDtypeStruct((_NW * _L,), jnp.float32),
        mesh=mesh,
        scratch_types=[
            pltpu.VMEM((2, joints, 2, chunk), jnp.float32),
            pltpu.VMEM((2, joints, 2, chunk), jnp.float32),
            pltpu.VMEM((2, joints, chunk), jnp.float32),
            pltpu.VMEM((_L,), jnp.float32),
            pltpu.SemaphoreType.DMA,
            pltpu.SemaphoreType.DMA,
        ],
        compiler_params=pltpu.CompilerParams(
            use_tc_tiling_on_sc=False, needs_layout_passes=False),
        interpret=interpret,
    )
    def sc_kernel(o_hbm, t_hbm, w_hbm, out_hbm, o_v, t_v, w_v, acc_v,
                  sem0, sem1):
        cid = lax.axis_index("c")
        sid = lax.axis_index("s")
        wid = sid * _NC + cid
        b0 = wid * rows_per_w
        sems = (sem0, sem1)
        acc_v[...] = jnp.zeros((_L,), jnp.float32)

        def copies(ci, buf):
            b = b0 + ci * chunk
            g = b // _BL
            l0 = b % _BL
            return (
                pltpu.make_async_copy(
                    o_hbm.at[:, g, :, pl.ds(l0, chunk)], o_v.at[buf],
                    sems[buf]),
                pltpu.make_async_copy(
                    t_hbm.at[:, g, :, pl.ds(l0, chunk)], t_v.at[buf],
                    sems[buf]),
                pltpu.make_async_copy(
                    w_hbm.at[:, pl.ds(b, chunk)], w_v.at[buf], sems[buf]),
            )

        def start(ci, buf):
            for c in copies(ci, buf):
                c.start()

        def wait(ci, buf):
            for c in copies(ci, buf):
                c.wait()

        def process(buf):
            neg = jnp.full((_L,), _NEG, jnp.float32)

            for gi in range(0, ngrp, 2):
                sa = gi * _L
                sb = (gi + 1) * _L

                def jbody(j, ms):
                    a1, a2, a3, a4, a5, b1, b2, b3, b4, b5 = ms
                    oa0 = o_v[buf, j, 0, pl.ds(sa, _L)]
                    oa1 = o_v[buf, j, 1, pl.ds(sa, _L)]
                    ta0 = t_v[buf, j, 0, pl.ds(sa, _L)]
                    ta1 = t_v[buf, j, 1, pl.ds(sa, _L)]
                    wa = w_v[buf, j, pl.ds(sa, _L)]
                    ob0 = o_v[buf, j, 0, pl.ds(sb, _L)]
                    ob1 = o_v[buf, j, 1, pl.ds(sb, _L)]
                    tb0 = t_v[buf, j, 0, pl.ds(sb, _L)]
                    tb1 = t_v[buf, j, 1, pl.ds(sb, _L)]
                    wb = w_v[buf, j, pl.ds(sb, _L)]
                    da0 = oa0 - ta0
                    da1 = oa1 - ta1
                    va = (da0 * da0 + da1 * da1) * wa
                    db0 = ob0 - tb0
                    db1 = ob1 - tb1
                    vb = (db0 * db0 + db1 * db1) * wb
                    n1 = jnp.maximum(a1, va)
                    r = jnp.minimum(a1, va)
                    n2 = jnp.maximum(a2, r)
                    r = jnp.minimum(a2, r)
                    n3 = jnp.maximum(a3, r)
                    r = jnp.minimum(a3, r)
                    n4 = jnp.maximum(a4, r)
                    r = jnp.minimum(a4, r)
                    n5 = jnp.maximum(a5, r)
                    p1 = jnp.maximum(b1, vb)
                    q = jnp.minimum(b1, vb)
                    p2 = jnp.maximum(b2, q)
                    q = jnp.minimum(b2, q)
                    p3 = jnp.maximum(b3, q)
                    q = jnp.minimum(b3, q)
                    p4 = jnp.maximum(b4, q)
                    q = jnp.minimum(b4, q)
                    p5 = jnp.maximum(b5, q)
                    return (n1, n2, n3, n4, n5, p1, p2, p3, p4, p5)

                ms = lax.fori_loop(0, joints, jbody, (neg,) * 10)
                a1, a2, a3, a4, a5, b1, b2, b3, b4, b5 = ms
                acc_v[...] = (acc_v[...] + (a1 + a2 + a3 + a4 + a5)
                              + (b1 + b2 + b3 + b4 + b5))

        start(0, 0)

        def pipe_body(k, carry):
            ca = 2 * k
            start(ca + 1, 1)
            wait(ca, 0)
            process(0)

            @pl.when(k < (nchunk // 2) - 1)
            def _():
                start(ca + 2, 0)

            wait(ca + 1, 1)
            process(1)
            return carry

        lax.fori_loop(0, nchunk // 2, pipe_body, 0)
        pltpu.sync_copy(acc_v, out_hbm.at[pl.ds(wid * _L, _L)])

    return sc_kernel


def kernel(output, target, target_weight):
    batch, joints, _ = output.shape
    o4 = output.reshape(_BL, batch // _BL, joints, 2).transpose(2, 0, 3, 1)
    t4 = target.reshape(_BL, batch // _BL, joints, 2).transpose(2, 0, 3, 1)
    wt = target_weight.T
    parts = _build(batch, joints)(o4, t4, wt)
    return jnp.sum(parts) * (1.0 / (_TOPK * batch))


# speedup vs baseline: 7.6114x; 1.0030x over previous
import functools

import jax
import jax.numpy as jnp
from jax import lax
from jax.experimental import pallas as pl
from jax.experimental.pallas import tpu as pltpu
from jax.experimental.pallas import tpu_sc as plsc

_TOPK = 5
_NC = 2
_NS = 16
_NW = _NC * _NS
_L = 16
_BL = 128

_NEG = float(jnp.finfo(jnp.float32).min)


@functools.lru_cache(maxsize=None)
def _build(batch: int, joints: int, interpret: bool = False):
    rows_per_w = batch // _NW
    chunk = 64
    nchunk = rows_per_w // chunk
    ngrp = chunk // _L

    mesh = plsc.VectorSubcoreMesh(
        core_axis_name="c", subcore_axis_name="s", num_cores=_NC,
        num_subcores=_NS)

    @functools.partial(
        pl.kernel,
        out_type=jax.ShapeDtypeStruct((_NW * _L,), jnp.float32),
        mesh=mesh,
        scratch_types=[
            pltpu.VMEM((2, joints, 2, chunk), jnp.float32),
            pltpu.VMEM((2, joints, 2, chunk), jnp.float32),
            pltpu.VMEM((2, joints, chunk), jnp.float32),
            pltpu.VMEM((_L,), jnp.float32),
            pltpu.SemaphoreType.DMA,
            pltpu.SemaphoreType.DMA,
        ],
        compiler_params=pltpu.CompilerParams(
            use_tc_tiling_on_sc=False, needs_layout_passes=False),
        interpret=interpret,
    )
    def sc_kernel(o_hbm, t_hbm, w_hbm, out_hbm, o_v, t_v, w_v, acc_v,
                  sem0, sem1):
        cid = lax.axis_index("c")
        sid = lax.axis_index("s")
        wid = sid * _NC + cid
        b0 = wid * rows_per_w
        sems = (sem0, sem1)
        acc_v[...] = jnp.zeros((_L,), jnp.float32)

        def copies(ci, buf):
            b = b0 + ci * chunk
            g = b // _BL
            l0 = b % _BL
            return (
                pltpu.make_async_copy(
                    o_hbm.at[:, g, :, pl.ds(l0, chunk)], o_v.at[buf],
                    sems[buf]),
                pltpu.make_async_copy(
                    t_hbm.at[:, g, :, pl.ds(l0, chunk)], t_v.at[buf],
                    sems[buf]),
                pltpu.make_async_copy(
                    w_hbm.at[:, pl.ds(b, chunk)], w_v.at[buf], sems[buf]),
            )

        def start(ci, buf):
            for c in copies(ci, buf):
                c.start()

        def wait(ci, buf):
            for c in copies(ci, buf):
                c.wait()

        def process(buf):
            neg = jnp.full((_L,), _NEG, jnp.float32)

            def jbody(j, ms):
                out = []
                for gi in range(ngrp):
                    s = gi * _L
                    m1, m2, m3, m4, m5 = ms[5 * gi:5 * gi + 5]
                    o0 = o_v[buf, j, 0, pl.ds(s, _L)]
                    o1 = o_v[buf, j, 1, pl.ds(s, _L)]
                    t0 = t_v[buf, j, 0, pl.ds(s, _L)]
                    t1 = t_v[buf, j, 1, pl.ds(s, _L)]
                    tw = w_v[buf, j, pl.ds(s, _L)]
                    d0 = o0 - t0
                    d1 = o1 - t1
                    v = (d0 * d0 + d1 * d1) * tw
                    n1 = jnp.maximum(m1, v)
                    r = jnp.minimum(m1, v)
                    n2 = jnp.maximum(m2, r)
                    r = jnp.minimum(m2, r)
                    n3 = jnp.maximum(m3, r)
                    r = jnp.minimum(m3, r)
                    n4 = jnp.maximum(m4, r)
                    r = jnp.minimum(m4, r)
                    n5 = jnp.maximum(m5, r)
                    out += [n1, n2, n3, n4, n5]
                return tuple(out)

            ms = lax.fori_loop(0, joints, jbody, (neg,) * (5 * ngrp))
            tot = acc_v[...]
            for gi in range(ngrp):
                m1, m2, m3, m4, m5 = ms[5 * gi:5 * gi + 5]
                tot = tot + (m1 + m2 + m3 + m4 + m5)
            acc_v[...] = tot

        start(0, 0)

        def pipe_body(k, carry):
            ca = 2 * k
            start(ca + 1, 1)
            wait(ca, 0)
            process(0)

            @pl.when(k < (nchunk // 2) - 1)
            def _():
                start(ca + 2, 0)

            wait(ca + 1, 1)
            process(1)
            return carry

        lax.fori_loop(0, nchunk // 2, pipe_body, 0)
        pltpu.sync_copy(acc_v, out_hbm.at[pl.ds(wid * _L, _L)])

    return sc_kernel


def kernel(output, target, target_weight):
    batch, joints, _ = output.shape
    o4 = output.reshape(_BL, batch // _BL, joints, 2).transpose(2, 0, 3, 1)
    t4 = target.reshape(_BL, batch // _BL, joints, 2).transpose(2, 0, 3, 1)
    wt = target_weight.T
    parts = _build(batch, joints)(o4, t4, wt)
    return jnp.sum(parts) * (1.0 / (_TOPK * batch))


# speedup vs baseline: 7.9424x; 1.0435x over previous
import functools

import jax
import jax.numpy as jnp
from jax import lax
from jax.experimental import pallas as pl
from jax.experimental.pallas import tpu as pltpu
from jax.experimental.pallas import tpu_sc as plsc

_TOPK = 5
_NC = 2
_NS = 16
_NW = _NC * _NS
_L = 16
_BL = 128

_NEG = float(jnp.finfo(jnp.float32).min)


@functools.lru_cache(maxsize=None)
def _build(batch: int, joints: int, interpret: bool = False):
    rows_per_w = batch // _NW
    chunk = 64
    nchunk = rows_per_w // chunk
    ngrp = chunk // _L

    mesh = plsc.VectorSubcoreMesh(
        core_axis_name="c", subcore_axis_name="s", num_cores=_NC,
        num_subcores=_NS)

    @functools.partial(
        pl.kernel,
        out_type=jax.ShapeDtypeStruct((_NW * _L,), jnp.float32),
        mesh=mesh,
        scratch_types=[
            pltpu.VMEM((2, 2, joints, chunk), jnp.float32),
            pltpu.VMEM((2, 2, joints, chunk), jnp.float32),
            pltpu.VMEM((2, joints, chunk), jnp.float32),
            pltpu.VMEM((_L,), jnp.float32),
            pltpu.SemaphoreType.DMA,
            pltpu.SemaphoreType.DMA,
        ],
        compiler_params=pltpu.CompilerParams(
            use_tc_tiling_on_sc=False, needs_layout_passes=False),
        interpret=interpret,
    )
    def sc_kernel(o_hbm, t_hbm, w_hbm, out_hbm, o_v, t_v, w_v, acc_v,
                  sem0, sem1):
        cid = lax.axis_index("c")
        sid = lax.axis_index("s")
        wid = sid * _NC + cid
        b0 = wid * rows_per_w
        sems = (sem0, sem1)
        acc_v[...] = jnp.zeros((_L,), jnp.float32)

        def copies(ci, buf):
            b = b0 + ci * chunk
            g = b // _BL
            l0 = b % _BL
            return (
                pltpu.make_async_copy(
                    o_hbm.at[:, g, 0, pl.ds(l0, chunk)], o_v.at[buf, 0],
                    sems[buf]),
                pltpu.make_async_copy(
                    o_hbm.at[:, g, 1, pl.ds(l0, chunk)], o_v.at[buf, 1],
                    sems[buf]),
                pltpu.make_async_copy(
                    t_hbm.at[:, g, 0, pl.ds(l0, chunk)], t_v.at[buf, 0],
                    sems[buf]),
                pltpu.make_async_copy(
                    t_hbm.at[:, g, 1, pl.ds(l0, chunk)], t_v.at[buf, 1],
                    sems[buf]),
                pltpu.make_async_copy(
                    w_hbm.at[:, pl.ds(b, chunk)], w_v.at[buf], sems[buf]),
            )

        def start(ci, buf):
            for c in copies(ci, buf):
                c.start()

        def wait(ci, buf):
            for c in copies(ci, buf):
                c.wait()

        def process(buf):
            neg = jnp.full((_L,), _NEG, jnp.float32)

            def jbody(j, ms):
                out = []
                for gi in range(ngrp):
                    s = gi * _L
                    m1, m2, m3, m4, m5 = ms[5 * gi:5 * gi + 5]
                    o0 = o_v[buf, 0, j, pl.ds(s, _L)]
                    o1 = o_v[buf, 1, j, pl.ds(s, _L)]
                    t0 = t_v[buf, 0, j, pl.ds(s, _L)]
                    t1 = t_v[buf, 1, j, pl.ds(s, _L)]
                    tw = w_v[buf, j, pl.ds(s, _L)]
                    d0 = o0 - t0
                    d1 = o1 - t1
                    v = (d0 * d0 + d1 * d1) * tw
                    n1 = jnp.maximum(m1, v)
                    r = jnp.minimum(m1, v)
                    n2 = jnp.maximum(m2, r)
                    r = jnp.minimum(m2, r)
                    n3 = jnp.maximum(m3, r)
                    r = jnp.minimum(m3, r)
                    n4 = jnp.maximum(m4, r)
                    r = jnp.minimum(m4, r)
                    n5 = jnp.maximum(m5, r)
                    out += [n1, n2, n3, n4, n5]
                return tuple(out)

            ms = lax.fori_loop(0, joints, jbody, (neg,) * (5 * ngrp))
            tot = acc_v[...]
            for gi in range(ngrp):
                m1, m2, m3, m4, m5 = ms[5 * gi:5 * gi + 5]
                tot = tot + (m1 + m2 + m3 + m4 + m5)
            acc_v[...] = tot

        start(0, 0)

        def pipe_body(k, carry):
            ca = 2 * k
            start(ca + 1, 1)
            wait(ca, 0)
            process(0)

            @pl.when(k < (nchunk // 2) - 1)
            def _():
                start(ca + 2, 0)

            wait(ca + 1, 1)
            process(1)
            return carry

        lax.fori_loop(0, nchunk // 2, pipe_body, 0)
        pltpu.sync_copy(acc_v, out_hbm.at[pl.ds(wid * _L, _L)])

    return sc_kernel


def kernel(output, target, target_weight):
    batch, joints, _ = output.shape
    o4 = output.reshape(_BL, batch // _BL, joints, 2).transpose(2, 0, 3, 1)
    t4 = target.reshape(_BL, batch // _BL, joints, 2).transpose(2, 0, 3, 1)
    wt = target_weight.T
    parts = _build(batch, joints)(o4, t4, wt)
    return jnp.sum(parts) * (1.0 / (_TOPK * batch))
